# baseline (speedup 1.0000x reference)
import functools

import jax
import jax.numpy as jnp
from jax import lax
from jax.experimental import pallas as pl
from jax.experimental.pallas import tpu as pltpu
from jax.experimental.pallas import tpu_sc as plsc

B, N, K, D = 4, 4096, 16, 128
BN = B * N
BNK = B * N * K
NW = 32
ROWS_PW = BNK // NW
CH2 = 128
NPAIR = ROWS_PW // CH2 // 2
SQRT_MAGIC = 0x1FBD1DF5


def _wprep_body(w_ref, b_ref, o_ref):
    w = w_ref[...]
    o_ref[...] = jnp.concatenate(
        [w[0:3] - w[6:9], b_ref[...], w[3:6] + w[6:9], w[9:10]], axis=0)


def _wprep(W, b2d):
    return pl.pallas_call(
        _wprep_body,
        out_shape=jax.ShapeDtypeStruct((8, D), jnp.float32),
    )(W, b2d)


def _sqrt16(x):
    i = plsc.bitcast(x, jnp.int32)
    y = plsc.bitcast((i >> 1) + SQRT_MAGIC, jnp.float32)
    y = 0.5 * (y + x / y)
    y = 0.5 * (y + x / y)
    return y


def _splat(v, lane):
    return lax.gather(
        v, jnp.full((16, 1), lane, jnp.int32),
        lax.GatherDimensionNumbers(
            offset_dims=(), collapsed_slice_dims=(0,), start_index_map=(0,)),
        (1,), mode=lax.GatherScatterMode.PROMISE_IN_BOUNDS)


def _sc_fuse_body(feat_hbm, wtab_hbm, tx_hbm, ty_hbm, tz_hbm, gidx_hbm,
                  out_hbm,
                  txv, tyv, tzv, wv, idxa, idxb, tba, tbb, rba, rbb,
                  sra, srb, sfa, sfb,
                  sem_i, sem_ta, sem_tb, sem_oa, sem_ob, sem_ra, sem_rb):
    wid = lax.axis_index("s") * 2 + lax.axis_index("c")
    base0 = wid * ROWS_PW
    pltpu.sync_copy(tx_hbm, txv)
    pltpu.sync_copy(ty_hbm, tyv)
    pltpu.sync_copy(tz_hbm, tzv)
    pltpu.sync_copy(wtab_hbm, wv)
    wc0 = [wv[4, pl.ds(l * 16, 16)] for l in range(8)]
    wc1 = [wv[5, pl.ds(l * 16, 16)] for l in range(8)]
    wc2 = [wv[6, pl.ds(l * 16, 16)] for l in range(8)]
    w9 = [wv[7, pl.ds(l * 16, 16)] for l in range(8)]

    def fuse_chunk(chunk_base, idxv, tbuf, rbuf, sidr, sidf):
        pt0 = chunk_base // K

        def sbuild(i, c):
            s = pl.ds(i * 16, 16)
            rows = chunk_base + i * 16 + lax.iota(jnp.int32, 16)
            sidr[s] = rows * 2
            sidf[s] = rows * 2 + 1
            return c

        lax.fori_loop(0, CH2 // 16, sbuild, 0)

        def point(i, c):
            v = idxv[pl.ds(i * K, 16)]
            pxv = plsc.load_gather(txv, [v])
            pyv = plsc.load_gather(tyv, [v])
            pzv = plsc.load_gather(tzv, [v])
            csp = jnp.full((16,), pt0 + i, jnp.int32)
            cxv = plsc.load_gather(txv, [csp])
            cyv = plsc.load_gather(tyv, [csp])
            czv = plsc.load_gather(tzv, [csp])
            dx = pxv - cxv
            dy = pyv - cyv
            dz = pzv - czv
            norm16 = _sqrt16(dx * dx + dy * dy + dz * dz)
            g0pt = [cxv * wv[0, pl.ds(l * 16, 16)]
                    + cyv * wv[1, pl.ds(l * 16, 16)]
                    + czv * wv[2, pl.ds(l * 16, 16)]
                    + wv[3, pl.ds(l * 16, 16)] for l in range(8)]
            for jj in range(K):
                r = i * K + jj
                pxs = _splat(pxv, jj)
                pys = _splat(pyv, jj)
                pzs = _splat(pzv, jj)
                nb = _splat(norm16, jj)
                for l in range(8):
                    s = pl.ds(l * 16, 16)
                    val = (g0pt[l] + pxs * wc0[l]
                           + pys * wc1[l] + pzs * wc2[l] + nb * w9[l])
                    rbuf[r, s] = jnp.maximum(val, 0.0)
            return c

        lax.fori_loop(0, CH2 // K, point, 0)

    pltpu.async_copy(gidx_hbm.at[pl.ds(base0, CH2)], idxa, sem_i).wait()
    pltpu.async_copy(feat_hbm.at[idxa], tba, sem_ta)
    pltpu.async_copy(gidx_hbm.at[pl.ds(base0 + CH2, CH2)], idxb, sem_i).wait()
    pltpu.async_copy(feat_hbm.at[idxb], tbb, sem_tb)

    def pair(jj, carry):
        b0 = base0 + (2 * jj) * CH2
        b1 = b0 + CH2
        pltpu.make_async_copy(feat_hbm.at[idxa], tba, sem_ta).wait()
        fuse_chunk(b0, idxa, tba, rba, sra, sfa)
        pltpu.async_copy(rba, out_hbm.at[sra], sem_ra)
        pltpu.async_copy(tba, out_hbm.at[sfa], sem_oa)
        pltpu.make_async_copy(feat_hbm.at[idxb], tbb, sem_tb).wait()
        fuse_chunk(b1, idxb, tbb, rbb, srb, sfb)
        pltpu.async_copy(rbb, out_hbm.at[srb], sem_rb)
        pltpu.async_copy(tbb, out_hbm.at[sfb], sem_ob)

        @pl.when(jj + 1 < NPAIR)
        def _():
            b2 = b1 + CH2
            b3 = b2 + CH2
            pltpu.async_copy(gidx_hbm.at[pl.ds(b2, CH2)], idxa, sem_i).wait()
            pltpu.make_async_copy(rba, out_hbm.at[sra], sem_ra).wait()
            pltpu.make_async_copy(tba, out_hbm.at[sfa], sem_oa).wait()
            pltpu.async_copy(feat_hbm.at[idxa], tba, sem_ta)
            pltpu.async_copy(gidx_hbm.at[pl.ds(b3, CH2)], idxb, sem_i).wait()
            pltpu.make_async_copy(rbb, out_hbm.at[srb], sem_rb).wait()
            pltpu.make_async_copy(tbb, out_hbm.at[sfb], sem_ob).wait()
            pltpu.async_copy(feat_hbm.at[idxb], tbb, sem_tb)

        return carry

    lax.fori_loop(0, NPAIR, pair, 0)
    pltpu.make_async_copy(rba, out_hbm.at[sra], sem_ra).wait()
    pltpu.make_async_copy(tba, out_hbm.at[sfa], sem_oa).wait()
    pltpu.make_async_copy(rbb, out_hbm.at[srb], sem_rb).wait()
    pltpu.make_async_copy(tbb, out_hbm.at[sfb], sem_ob).wait()


def _sc_fuse(feat2d, wtab, tx, ty, tz, gidx):
    mesh = plsc.VectorSubcoreMesh(core_axis_name="c", subcore_axis_name="s")
    fn = functools.partial(
        pl.kernel,
        mesh=mesh,
        compiler_params=pltpu.CompilerParams(needs_layout_passes=False),
        out_type=jax.ShapeDtypeStruct((2 * BNK, D), jnp.float32),
        scratch_types=[
            pltpu.VMEM((BN,), jnp.float32),
            pltpu.VMEM((BN,), jnp.float32),
            pltpu.VMEM((BN,), jnp.float32),
            pltpu.VMEM((8, D), jnp.float32),
            pltpu.VMEM((CH2,), jnp.int32),
            pltpu.VMEM((CH2,), jnp.int32),
            pltpu.VMEM((CH2, D), jnp.float32),
            pltpu.VMEM((CH2, D), jnp.float32),
            pltpu.VMEM((CH2, D), jnp.float32),
            pltpu.VMEM((CH2, D), jnp.float32),
            pltpu.VMEM((CH2,), jnp.int32),
            pltpu.VMEM((CH2,), jnp.int32),
            pltpu.VMEM((CH2,), jnp.int32),
            pltpu.VMEM((CH2,), jnp.int32),
            pltpu.SemaphoreType.DMA,
            pltpu.SemaphoreType.DMA,
            pltpu.SemaphoreType.DMA,
            pltpu.SemaphoreType.DMA,
            pltpu.SemaphoreType.DMA,
            pltpu.SemaphoreType.DMA,
            pltpu.SemaphoreType.DMA,
        ],
    )(_sc_fuse_body)
    return fn(feat2d, wtab, tx, ty, tz, gidx)


def kernel(xyz, feat, idx, W, b):
    xyz2 = xyz.reshape(BN, 3)
    tx = xyz2[:, 0]
    ty = xyz2[:, 1]
    tz = xyz2[:, 2]
    feat2d = feat.reshape(BN, D)
    gidx = (idx + (jnp.arange(B, dtype=idx.dtype) * N)[:, None, None])
    gidx = gidx.reshape(BNK)
    wtab = _wprep(W, b.reshape(1, D))
    out = _sc_fuse(feat2d, wtab, tx, ty, tz, gidx)
    return out.reshape(B, N, K, 2 * D)


# speedup vs baseline: 1.8483x; 1.8483x over previous
import functools

import jax
import jax.numpy as jnp
from jax import lax
from jax.experimental import pallas as pl
from jax.experimental.pallas import tpu as pltpu
from jax.experimental.pallas import tpu_sc as plsc

B, N, K, D = 4, 4096, 16, 128
BN = B * N
BNK = B * N * K
NW = 32
ROWS_PW = BNK // NW
CH2 = 128
NCH = ROWS_PW // CH2
NPAIR = NCH // 2
PB = 1024
SQRT_MAGIC = 0x1FBD1DF5


def _tc_body(xyz_ref, f_ref, w_ref, b_ref, t_ref, g0_ref):
    w = w_ref[...]
    wa = w[0:3] - w[6:9]
    wc = w[3:6] + w[6:9]
    bb = b_ref[...]
    zpad = jnp.zeros((13, D), jnp.float32)
    wa16 = jnp.concatenate([wa, zpad], axis=0)
    wc16 = jnp.concatenate([wc, zpad], axis=0)

    cen = xyz_ref[...]
    dn = (((1,), (0,)), ((), ()))
    g1 = lax.dot_general(cen, wc16, dn, precision=lax.Precision.HIGHEST)
    g0 = lax.dot_general(cen, wa16, dn, precision=lax.Precision.HIGHEST) + bb
    t_ref[:, 0:D] = g1
    t_ref[:, D:2 * D] = f_ref[...]
    g0_ref[...] = g0


def _tc_tables(xyz16, feat2d, W, b2d):
    grid = (BN // PB,)
    return pl.pallas_call(
        _tc_body,
        grid=grid,
        in_specs=[
            pl.BlockSpec((PB, 16), lambda i: (i, 0)),
            pl.BlockSpec((PB, D), lambda i: (i, 0)),
            pl.BlockSpec((10, D), lambda i: (0, 0)),
            pl.BlockSpec((1, D), lambda i: (0, 0)),
        ],
        out_specs=[
            pl.BlockSpec((PB, 2 * D), lambda i: (i, 0)),
            pl.BlockSpec((PB, D), lambda i: (i, 0)),
        ],
        out_shape=[
            jax.ShapeDtypeStruct((BN, 2 * D), jnp.float32),
            jax.ShapeDtypeStruct((BN, D), jnp.float32),
        ],
    )(xyz16, feat2d, W, b2d)


def _sqrt16(x):
    i = plsc.bitcast(x, jnp.int32)
    y = plsc.bitcast((i >> 1) + SQRT_MAGIC, jnp.float32)
    y = 0.5 * (y + x / y)
    y = 0.5 * (y + x / y)
    return y


def _sc_fuse_body(t_hbm, g0_hbm, w9_hbm, tx_hbm, ty_hbm, tz_hbm, gidx_hbm,
                  out_hbm,
                  txv, tyv, tzv, idxa, idxb, tba, tbb, g0b, nrmt, w9v,
                  sem_i, sem_ta, sem_tb, sem_oa, sem_ob):
    wid = lax.axis_index("s") * 2 + lax.axis_index("c")
    base0 = wid * ROWS_PW
    pltpu.sync_copy(tx_hbm, txv)
    pltpu.sync_copy(ty_hbm, tyv)
    pltpu.sync_copy(tz_hbm, tzv)
    pltpu.sync_copy(w9_hbm, w9v)
    w9r = [w9v[pl.ds(l * 16, 16)] for l in range(8)]

    def fuse_chunk(chunk_base, idxv, tbuf):
        g0_off = pl.multiple_of(chunk_base // K, CH2 // K)
        pltpu.sync_copy(g0_hbm.at[pl.ds(g0_off, CH2 // K)], g0b)

        def point(i, c):
            v = idxv[pl.ds(i * K, 16)]
            pxv = plsc.load_gather(txv, [v])
            pyv = plsc.load_gather(tyv, [v])
            pzv = plsc.load_gather(tzv, [v])
            csp = jnp.full((16,), g0_off + i, jnp.int32)
            cxv = plsc.load_gather(txv, [csp])
            cyv = plsc.load_gather(tyv, [csp])
            czv = plsc.load_gather(tzv, [csp])
            dx = pxv - cxv
            dy = pyv - cyv
            dz = pzv - czv
            norm16 = _sqrt16(dx * dx + dy * dy + dz * dz)
            g0r = [g0b[i, pl.ds(l * 16, 16)] for l in range(8)]
            for jj in range(K):
                r = i * K + jj
                nb = lax.gather(
                    norm16, jnp.full((16, 1), jj, jnp.int32),
                    lax.GatherDimensionNumbers(
                        offset_dims=(), collapsed_slice_dims=(0,),
                        start_index_map=(0,)),
                    (1,), mode=lax.GatherScatterMode.PROMISE_IN_BOUNDS)
                for l in range(8):
                    s = pl.ds(l * 16, 16)
                    val = tbuf[r, s] + g0r[l] + nb * w9r[l]
                    tbuf[r, s] = jnp.maximum(val, 0.0)
            return c

        lax.fori_loop(0, CH2 // K, point, 0)

    pltpu.async_copy(gidx_hbm.at[pl.ds(base0, CH2)], idxa, sem_i).wait()
    pltpu.async_copy(t_hbm.at[idxa], tba, sem_ta)
    pltpu.async_copy(gidx_hbm.at[pl.ds(base0 + CH2, CH2)], idxb, sem_i).wait()
    pltpu.async_copy(t_hbm.at[idxb], tbb, sem_tb)

    def pair(jj, carry):
        b0 = base0 + (2 * jj) * CH2
        b1 = b0 + CH2
        pltpu.make_async_copy(t_hbm.at[idxa], tba, sem_ta).wait()
        fuse_chunk(b0, idxa, tba)
        pltpu.async_copy(tba, out_hbm.at[pl.ds(b0, CH2)], sem_oa)
        pltpu.make_async_copy(t_hbm.at[idxb], tbb, sem_tb).wait()
        fuse_chunk(b1, idxb, tbb)
        pltpu.async_copy(tbb, out_hbm.at[pl.ds(b1, CH2)], sem_ob)

        @pl.when(jj + 1 < NPAIR)
        def _():
            b2 = b1 + CH2
            b3 = b2 + CH2
            pltpu.async_copy(gidx_hbm.at[pl.ds(b2, CH2)], idxa, sem_i).wait()
            pltpu.make_async_copy(tba, out_hbm.at[pl.ds(b0, CH2)],
                                  sem_oa).wait()
            pltpu.async_copy(t_hbm.at[idxa], tba, sem_ta)
            pltpu.async_copy(gidx_hbm.at[pl.ds(b3, CH2)], idxb, sem_i).wait()
            pltpu.make_async_copy(tbb, out_hbm.at[pl.ds(b1, CH2)],
                                  sem_ob).wait()
            pltpu.async_copy(t_hbm.at[idxb], tbb, sem_tb)

        return carry

    lax.fori_loop(0, NPAIR, pair, 0)
    pltpu.make_async_copy(tba, out_hbm.at[pl.ds(base0, CH2)], sem_oa).wait()
    pltpu.make_async_copy(tbb, out_hbm.at[pl.ds(base0, CH2)], sem_ob).wait()


def _sc_fuse(t_tab, g0_tab, w9, tx, ty, tz, gidx):
    mesh = plsc.VectorSubcoreMesh(core_axis_name="c", subcore_axis_name="s")
    fn = functools.partial(
        pl.kernel,
        mesh=mesh,
        compiler_params=pltpu.CompilerParams(needs_layout_passes=False),
        out_type=jax.ShapeDtypeStruct((BNK, 2 * D), jnp.float32),
        scratch_types=[
            pltpu.VMEM((BN,), jnp.float32),
            pltpu.VMEM((BN,), jnp.float32),
            pltpu.VMEM((BN,), jnp.float32),
            pltpu.VMEM((CH2,), jnp.int32),
            pltpu.VMEM((CH2,), jnp.int32),
            pltpu.VMEM((CH2, 2 * D), jnp.float32),
            pltpu.VMEM((CH2, 2 * D), jnp.float32),
            pltpu.VMEM((CH2 // K, D), jnp.float32),
            pltpu.VMEM((16,), jnp.float32),
            pltpu.VMEM((D,), jnp.float32),
            pltpu.SemaphoreType.DMA,
            pltpu.SemaphoreType.DMA,
            pltpu.SemaphoreType.DMA,
            pltpu.SemaphoreType.DMA,
            pltpu.SemaphoreType.DMA,
        ],
    )(_sc_fuse_body)
    return fn(t_tab, g0_tab, w9, tx, ty, tz, gidx)


def kernel(xyz, feat, idx, W, b):
    xyz2 = xyz.reshape(BN, 3)
    xyz16 = jnp.pad(xyz2, ((0, 0), (0, 13)))
    tx = xyz2[:, 0]
    ty = xyz2[:, 1]
    tz = xyz2[:, 2]
    feat2d = feat.reshape(BN, D)
    gidx = (idx + (jnp.arange(B, dtype=idx.dtype) * N)[:, None, None])
    gidx = gidx.reshape(BNK)
    t_tab, g0_tab = _tc_tables(xyz16, feat2d, W, b.reshape(1, D))
    out = _sc_fuse(t_tab, g0_tab, W[9], tx, ty, tz, gidx)
    return out.reshape(B, N, K, 2 * D)
